# baseline (speedup 1.0000x reference)
import functools

import jax
import jax.numpy as jnp
from jax import lax
from jax.experimental import pallas as pl
from jax.experimental.pallas import tpu as pltpu


def _mega_kernel(x_ref, m_ref, w_ref, pmask_ref, bmask_ref, g_ref, bb_ref,
                 o_ref,
                 y_all, ps_ref, psq_ref,
                 xpad_ref, hpad_ref, hi16_ref, oh_ref, fb_ref,
                 *, block_num, width, chunk):
    C = x_ref.shape[1]
    H, Wd = x_ref.shape[2], x_ref.shape[3]
    P = H * Wd
    B = y_all.shape[0]
    F = y_all.shape[1]
    K = block_num
    W = width
    f32 = jnp.float32
    bf16 = jnp.bfloat16
    pad = W

    p = pl.program_id(0)
    b = pl.program_id(1)

    @pl.when(jnp.logical_and(p == 0, b == 0))
    def _zero_pads():
        xpad_ref[:, 0:pad] = jnp.zeros((C, pad), f32)
        xpad_ref[:, pad + P:] = jnp.zeros((C, W), f32)
        hpad_ref[:, 0:pad] = jnp.zeros((C, pad), f32)
        hpad_ref[:, pad + P:] = jnp.zeros((C, W), f32)
        zpadh = jnp.zeros((3 * C, pad), bf16)
        fb_ref[:, 0:pad] = zpadh
        fb_ref[:, pad + P:] = zpadh

    @pl.when(p == 0)
    def _compute_phase():
        xpad_ref[:, pad:pad + P] = x_ref[0].reshape(C, P)
        x = xpad_ref[:, pad:pad + P]

        ec = pmask_ref[0:1, :]
        er = pmask_ref[1:2, :]
        xl = xpad_ref[:, pad - 1:pad - 1 + P]
        xr = xpad_ref[:, pad + 1:pad + 1 + P]
        hsum = x + ec * xr + (1.0 - ec) * xl
        hpad_ref[:, pad:pad + P] = hsum
        hu = hpad_ref[:, 0:P]
        hd = hpad_ref[:, 2 * W:2 * W + P]
        vs = er * hd + (1.0 - er) * hu
        hi = x - 0.25 * (hpad_ref[:, pad:pad + P] + vs)

        mx = jnp.max(hi, axis=0, keepdims=True)
        hi16_ref[...] = hi[0:K]
        pref = jnp.full((1, P), -jnp.inf, f32)
        for c in range(K):
            hc = hi16_ref[c:c + 1, :]
            oh_ref[c:c + 1, :] = jnp.where((hc == mx) & (hc > pref), 1.0, 0.0)
            pref = jnp.maximum(pref, hc)
        onehot = oh_ref[...]

        sums = lax.dot_general(onehot, x, (((1,), (1,)), ((), ())),
                               preferred_element_type=f32)
        counts = jnp.sum(onehot, axis=1, keepdims=True)
        means = sums / (counts + (counts == 0).astype(f32))
        M = m_ref[...]
        q = jnp.dot(means, M, preferred_element_type=f32)
        g = lax.dot_general(q, means, (((1,), (1,)), ((), ())),
                            preferred_element_type=f32)
        eye = (lax.broadcasted_iota(jnp.int32, (K, K), 0) ==
               lax.broadcasted_iota(jnp.int32, (K, K), 1)).astype(f32)
        diag_col = jnp.sum(g * eye, axis=1, keepdims=True)
        diag_row = jnp.sum(g * eye, axis=0, keepdims=True)
        quad = diag_col + diag_row - 2.0 * g
        adj = jnp.exp(-quad) * (1.0 - eye)
        adjm = jnp.dot(adj, means, preferred_element_type=f32)
        feat = x + lax.dot_general(adjm, onehot, (((0,), (0,)), ((), ())),
                                   preferred_element_type=f32)

        fb_ref[C:2 * C, pad:pad + P] = feat.astype(bf16)
        fb_ref[0:C, pad:pad + P] = (fb_ref[C:2 * C, pad - 1:pad - 1 + P]
                                    * bmask_ref[0:1, :])
        fb_ref[2 * C:, pad:pad + P] = (fb_ref[C:2 * C, pad + 1:pad + 1 + P]
                                       * bmask_ref[1:2, :])

        bsum = jnp.zeros((F, 1), f32)
        bsq = jnp.zeros((F, 1), f32)
        for off in range(0, P, chunk):
            acc = jnp.zeros((F, chunk), f32)
            for t, di in enumerate((-1, 0, 1)):
                base = pad + di * W + off
                s = fb_ref[:, base:base + chunk]
                acc = acc + jnp.dot(w_ref[t], s,
                                    preferred_element_type=f32)
            y_all[b, :, off:off + chunk] = acc.astype(bf16)
            bsum = bsum + jnp.sum(acc, axis=1, keepdims=True)
            bsq = bsq + jnp.sum(acc * acc, axis=1, keepdims=True)
        zero = jnp.zeros((F, 1), f32)
        ps_ref[...] = jnp.where(b == 0, zero, ps_ref[...]) + bsum
        psq_ref[...] = jnp.where(b == 0, zero, psq_ref[...]) + bsq

    @pl.when(p == 1)
    def _bn_phase():
        n = B * P
        mean = ps_ref[...] * (1.0 / n)
        var = psq_ref[...] * (1.0 / n) - mean * mean
        inv = lax.rsqrt(var + 1e-5)
        scale = inv * g_ref[...]
        shift = bb_ref[...] - mean * scale
        rows = chunk // Wd
        for off in range(0, P, chunk):
            val = y_all[b, :, off:off + chunk].astype(f32) * scale + shift
            r0 = off // Wd
            o_ref[0, :, r0:r0 + rows, :] = val.reshape(F, rows, Wd)


def kernel(x, W, conv_w, bn_gamma, bn_beta):
    B, C, H, Wd = x.shape
    P = H * Wd
    K = 16
    F = conv_w.shape[0]
    f32 = jnp.float32
    bf16 = jnp.bfloat16
    chunk = 1024 if P % 1024 == 0 else P

    M = jnp.dot(W, W.T).astype(f32)
    w9 = (conv_w.transpose(2, 3, 0, 1).reshape(3, 3, F, C)
          .transpose(0, 2, 1, 3).reshape(3, F, 3 * C).astype(bf16))

    col = jnp.arange(P, dtype=jnp.int32) % Wd
    row = jnp.arange(P, dtype=jnp.int32) // Wd
    pmask = jnp.stack([(col % 2 == 0), (row % 2 == 0)]).astype(f32)
    bmask = jnp.stack([(col != 0), (col != Wd - 1)]).astype(bf16)

    kb = functools.partial(_mega_kernel, block_num=K, width=Wd, chunk=chunk)
    pe = P + 2 * Wd
    last = B - 1
    y_bn = pl.pallas_call(
        kb,
        out_shape=jax.ShapeDtypeStruct((B, F, H, Wd), f32),
        grid=(2, B),
        in_specs=[
            pl.BlockSpec((1, C, H, Wd),
                         lambda p, b: (b * (1 - p) + last * p, 0, 0, 0)),
            pl.BlockSpec((C, C), lambda p, b: (0, 0)),
            pl.BlockSpec((3, F, 3 * C), lambda p, b: (0, 0, 0)),
            pl.BlockSpec((2, P), lambda p, b: (0, 0)),
            pl.BlockSpec((2, P), lambda p, b: (0, 0)),
            pl.BlockSpec((F, 1), lambda p, b: (0, 0)),
            pl.BlockSpec((F, 1), lambda p, b: (0, 0)),
        ],
        out_specs=pl.BlockSpec((1, F, H, Wd), lambda p, b: (b * p, 0, 0, 0)),
        scratch_shapes=[pltpu.VMEM((B, F, P), bf16),
                        pltpu.VMEM((F, 1), f32),
                        pltpu.VMEM((F, 1), f32),
                        pltpu.VMEM((C, pe), f32),
                        pltpu.VMEM((C, pe), f32),
                        pltpu.VMEM((K, P), f32),
                        pltpu.VMEM((K, P), f32),
                        pltpu.VMEM((3 * C, pe), bf16)],
        compiler_params=pltpu.CompilerParams(
            dimension_semantics=("arbitrary", "arbitrary")),
    )(x, M, w9, pmask, bmask,
      bn_gamma.reshape(F, 1).astype(f32), bn_beta.reshape(F, 1).astype(f32))

    return y_bn
